# baseline (speedup 1.0000x reference)
import functools

import jax
import jax.numpy as jnp
from jax import lax
from jax.experimental import pallas as pl
from jax.experimental.pallas import tpu as pltpu
from jax.experimental.pallas import tpu_sc as plsc

B, V = 64, 100000
NEG_BIG = -3.0e38


CHUNK = 2048
NBLK = (V + CHUNK - 1) // CHUNK


def _tc_lse_kernel(x_ref, m_out, s_out, m_sc, s_sc):
    i = pl.program_id(0)

    @pl.when(i == 0)
    def _init():
        m_sc[...] = jnp.full_like(m_sc, NEG_BIG)
        s_sc[...] = jnp.zeros_like(s_sc)

    x = x_ref[...]

    @pl.when(i == NBLK - 1)
    def _mask_tail():
        col = i * CHUNK + lax.broadcasted_iota(jnp.int32, x.shape, 1)
        x_ref[...] = jnp.where(col < V, x, NEG_BIG)

    xm = x_ref[...]
    m_old = m_sc[...]
    m_new = jnp.maximum(m_old, jnp.max(xm, axis=1, keepdims=True))
    s_sc[...] = s_sc[...] * jnp.exp(m_old - m_new) + jnp.sum(
        jnp.exp(xm - m_new), axis=1, keepdims=True
    )
    m_sc[...] = m_new

    @pl.when(i == NBLK - 1)
    def _fin():
        m_out[...] = m_sc[...]
        s_out[...] = s_sc[...]


def _tc_lse(logits):
    return pl.pallas_call(
        _tc_lse_kernel,
        grid=(NBLK,),
        in_specs=[pl.BlockSpec((B, CHUNK), lambda i: (0, i))],
        out_specs=[
            pl.BlockSpec((B, 1), lambda i: (0, 0)),
            pl.BlockSpec((B, 1), lambda i: (0, 0)),
        ],
        out_shape=[
            jax.ShapeDtypeStruct((B, 1), jnp.float32),
            jax.ShapeDtypeStruct((B, 1), jnp.float32),
        ],
        scratch_shapes=[
            pltpu.VMEM((B, 1), jnp.float32),
            pltpu.VMEM((B, 1), jnp.float32),
        ],
    )(logits)


NC, NS, L = 2, 16, 16
NW = NC * NS
RPW = B // NW
C_ACT = 20000
NCH = V // C_ACT


@functools.partial(
    pl.kernel,
    out_type=[
        jax.ShapeDtypeStruct((NW, L), jnp.float32),
        jax.ShapeDtypeStruct((NW, L), jnp.float32),
    ],
    mesh=plsc.VectorSubcoreMesh(core_axis_name="c", subcore_axis_name="s"),
    scratch_types=[
        pltpu.VMEM((C_ACT,), jnp.float32),
        pltpu.VMEM((C_ACT,), jnp.float32),
        pltpu.VMEM((L,), jnp.float32),
        pltpu.VMEM((L,), jnp.float32),
        pltpu.SemaphoreType.DMA,
        pltpu.SemaphoreType.DMA,
        pltpu.SemaphoreType.DMA,
    ],
)
def _sc_scan(act_hbm, logf_hbm, d_out, n_out, ab0, ab1, gath, stage, sem0, sem1, semg):
    wid = lax.axis_index("s") * NC + lax.axis_index("c")
    r0 = wid * RPW
    bufs = (ab0, ab1)
    sems = (sem0, sem1)
    viota = lax.iota(jnp.int32, L)
    viota_f = lax.convert_element_type(viota, jnp.float32)
    zero16 = jnp.zeros((L,), jnp.float32)

    def start(t):
        row = r0 + (t // NCH)
        off = row * V + (t % NCH) * C_ACT
        return pltpu.async_copy(act_hbm.at[pl.ds(off, C_ACT)], bufs[t % 2], sems[t % 2])

    inflight = start(0)
    per_row = []
    vidx, vn = zero16, zero16
    for t in range(RPW * NCH):
        nxt = start(t + 1) if t + 1 < RPW * NCH else None
        inflight.wait()
        buf = bufs[t % 2]
        base = (t % NCH) * C_ACT

        def body(j, carry, buf=buf, base=base):
            vi, vc = carry
            a = buf[pl.ds(j * L, L)]
            colv = viota_f + lax.convert_element_type(base + j * L, jnp.float32)
            return vi + a * colv, vc + a

        vidx, vn = lax.fori_loop(0, C_ACT // L, body, (vidx, vn))
        if t % NCH == NCH - 1:
            per_row.append((vidx, vn))
            vidx, vn = zero16, zero16
        inflight = nxt

    def lane_sum(vec):
        acc = vec[0]
        for k in range(1, L):
            acc = acc + vec[k]
        return acc

    idx0 = lax.convert_element_type(lane_sum(per_row[0][0]), jnp.int32)
    idx1 = lax.convert_element_type(lane_sum(per_row[1][0]), jnp.int32)
    n0 = lane_sum(per_row[0][1])
    n1 = lane_sum(per_row[1][1])
    gvec = jnp.where(viota == 1, (r0 + 1) * V + idx1, r0 * V + idx0)
    pltpu.async_copy(logf_hbm.at[gvec], gath, semg).wait()
    nvec = jnp.where(viota == 0, n0, jnp.where(viota == 1, n1, 0.0))
    dvec = gath[...] * nvec
    stage[...] = dvec
    pltpu.sync_copy(stage, d_out.at[wid])
    stage[...] = nvec
    pltpu.sync_copy(stage, n_out.at[wid])


def kernel(logits, actions):
    m, s = _tc_lse(logits)
    d16, n16 = _sc_scan(actions.reshape(-1), logits.reshape(-1))
    d = d16[:, :RPW].reshape(B, 1)
    n = n16[:, :RPW].reshape(B, 1)
    lse = m + jnp.log(s)
    from jax.scipy.special import gammaln

    rt_zero = jnp.minimum(jnp.abs(s[0, 0]), jnp.float32(0.0))
    a0 = gammaln(jnp.float32(1.0) + rt_zero)
    a1 = gammaln(jnp.float32(2.0) + rt_zero)
    out = a1 + d - n * lse - (a0 * (V - n) + a1 * n)
    return out


# speedup vs baseline: 1.0928x; 1.0928x over previous
import functools

import jax
import jax.numpy as jnp
from jax import lax
from jax.experimental import pallas as pl
from jax.experimental.pallas import tpu as pltpu
from jax.experimental.pallas import tpu_sc as plsc

B, V = 64, 100000
NEG_BIG = -3.0e38


CHUNK = 2048
NBLK = (V + CHUNK - 1) // CHUNK


def _tc_lse_kernel(x_ref, m_out, s_out, m_sc, s_sc):
    i = pl.program_id(0)

    @pl.when(i == 0)
    def _init():
        m_sc[...] = jnp.full_like(m_sc, NEG_BIG)
        s_sc[...] = jnp.zeros_like(s_sc)

    x = x_ref[...]

    @pl.when(i == NBLK - 1)
    def _mask_tail():
        col = i * CHUNK + lax.broadcasted_iota(jnp.int32, x.shape, 1)
        x_ref[...] = jnp.where(col < V, x, NEG_BIG)

    xm = x_ref[...]
    m_old = m_sc[...]
    m_new = jnp.maximum(m_old, jnp.max(xm, axis=1, keepdims=True))
    s_sc[...] = s_sc[...] * jnp.exp(m_old - m_new) + jnp.sum(
        jnp.exp(xm - m_new), axis=1, keepdims=True
    )
    m_sc[...] = m_new

    @pl.when(i == NBLK - 1)
    def _fin():
        m_out[...] = m_sc[...]
        s_out[...] = s_sc[...]


def _tc_lse(logits):
    return pl.pallas_call(
        _tc_lse_kernel,
        grid=(NBLK,),
        in_specs=[pl.BlockSpec((B, CHUNK), lambda i: (0, i))],
        out_specs=[
            pl.BlockSpec((B, 1), lambda i: (0, 0)),
            pl.BlockSpec((B, 1), lambda i: (0, 0)),
        ],
        out_shape=[
            jax.ShapeDtypeStruct((B, 1), jnp.float32),
            jax.ShapeDtypeStruct((B, 1), jnp.float32),
        ],
        scratch_shapes=[
            pltpu.VMEM((B, 1), jnp.float32),
            pltpu.VMEM((B, 1), jnp.float32),
        ],
    )(logits)


NC, NS, L = 2, 16, 16
NW = NC * NS
RPW = B // NW
C_ACT = 50000
NCH = V // C_ACT


@functools.partial(
    pl.kernel,
    out_type=[
        jax.ShapeDtypeStruct((NW, L), jnp.float32),
        jax.ShapeDtypeStruct((NW, L), jnp.float32),
    ],
    mesh=plsc.VectorSubcoreMesh(core_axis_name="c", subcore_axis_name="s"),
    scratch_types=[
        pltpu.VMEM((C_ACT,), jnp.float32),
        pltpu.VMEM((C_ACT,), jnp.float32),
        pltpu.VMEM((L,), jnp.float32),
        pltpu.VMEM((L,), jnp.float32),
        pltpu.SemaphoreType.DMA,
        pltpu.SemaphoreType.DMA,
        pltpu.SemaphoreType.DMA,
    ],
)
def _sc_scan(act_hbm, logf_hbm, d_out, n_out, ab0, ab1, gath, stage, sem0, sem1, semg):
    wid = lax.axis_index("s") * NC + lax.axis_index("c")
    r0 = wid * RPW
    bufs = (ab0, ab1)
    sems = (sem0, sem1)
    viota = lax.iota(jnp.int32, L)
    viota_f = lax.convert_element_type(viota, jnp.float32)
    zero16 = jnp.zeros((L,), jnp.float32)

    def start(t):
        row = r0 + (t // NCH)
        off = row * V + (t % NCH) * C_ACT
        return pltpu.async_copy(act_hbm.at[pl.ds(off, C_ACT)], bufs[t % 2], sems[t % 2])

    step16 = jnp.full((L,), float(L), jnp.float32)
    UNROLL = 25
    inflight = start(0)
    per_row = []
    vidx, vn = zero16, zero16
    for t in range(RPW * NCH):
        nxt = start(t + 1) if t + 1 < RPW * NCH else None
        inflight.wait()
        buf = bufs[t % 2]
        base = (t % NCH) * C_ACT

        def body(j, carry, buf=buf, base=base):
            colv, vi, vc = carry
            off = j * (L * UNROLL)
            for u in range(UNROLL):
                a = buf[pl.ds(off + u * L, L)]
                vi = vi + a * colv
                vc = vc + a
                colv = colv + step16
            return colv, vi, vc

        colv0 = viota_f + float(base)
        _, vidx, vn = lax.fori_loop(
            0, C_ACT // (L * UNROLL), body, (colv0, vidx, vn)
        )
        if t % NCH == NCH - 1:
            per_row.append((vidx, vn))
            vidx, vn = zero16, zero16
        inflight = nxt

    def lane_sum(vec):
        acc = vec[0]
        for k in range(1, L):
            acc = acc + vec[k]
        return acc

    idx0 = lax.convert_element_type(lane_sum(per_row[0][0]), jnp.int32)
    idx1 = lax.convert_element_type(lane_sum(per_row[1][0]), jnp.int32)
    n0 = lane_sum(per_row[0][1])
    n1 = lane_sum(per_row[1][1])
    gvec = jnp.where(viota == 1, (r0 + 1) * V + idx1, r0 * V + idx0)
    pltpu.async_copy(logf_hbm.at[gvec], gath, semg).wait()
    nvec = jnp.where(viota == 0, n0, jnp.where(viota == 1, n1, 0.0))
    dvec = gath[...] * nvec
    stage[...] = dvec
    pltpu.sync_copy(stage, d_out.at[wid])
    stage[...] = nvec
    pltpu.sync_copy(stage, n_out.at[wid])


def kernel(logits, actions):
    m, s = _tc_lse(logits)
    d16, n16 = _sc_scan(actions.reshape(-1), logits.reshape(-1))
    d = d16[:, :RPW].reshape(B, 1)
    n = n16[:, :RPW].reshape(B, 1)
    lse = m + jnp.log(s)
    from jax.scipy.special import gammaln

    rt_zero = jnp.minimum(jnp.abs(s[0, 0]), jnp.float32(0.0))
    a0 = gammaln(jnp.float32(1.0) + rt_zero)
    a1 = gammaln(jnp.float32(2.0) + rt_zero)
    out = a1 + d - n * lse - (a0 * (V - n) + a1 * n)
    return out


# speedup vs baseline: 2.8630x; 2.6200x over previous
import functools

import jax
import jax.numpy as jnp
from jax import lax
from jax.experimental import pallas as pl
from jax.experimental.pallas import tpu as pltpu
from jax.experimental.pallas import tpu_sc as plsc

B, V = 64, 100000
NEG_BIG = -3.0e38

SC_T0 = 269
SC_NT = 512
V_TC = SC_T0 * 128
V_SC_END = (SC_T0 + SC_NT) * 128


TCW = 2048
TC_STEPS = V_TC // TCW + 2


def _tc_kernel(x_ref, a_ref, m_out, s_out, d_out, n_out, m_sc, s_sc, d_sc, n_sc):
    i = pl.program_id(0)

    @pl.when(i == 0)
    def _init():
        m_sc[...] = jnp.full_like(m_sc, NEG_BIG)
        s_sc[...] = jnp.zeros_like(s_sc)
        d_sc[...] = jnp.zeros_like(d_sc)
        n_sc[...] = jnp.zeros_like(n_sc)

    x = x_ref[...]
    a = a_ref[...]
    blk = jnp.where(i == TC_STEPS - 1, (V_SC_END // TCW), i)
    col = blk * TCW + lax.broadcasted_iota(jnp.int32, x.shape, 1)
    hi = jnp.where(i == TC_STEPS - 1, V, V_TC)
    lo = jnp.where(i == TC_STEPS - 1, V_SC_END, 0)
    mask = (col >= lo) & (col < hi)
    xm = jnp.where(mask, x, NEG_BIG)

    m_old = m_sc[...]
    m_new = jnp.maximum(m_old, jnp.max(xm, axis=1, keepdims=True))
    s_sc[...] = s_sc[...] * jnp.exp(m_old - m_new) + jnp.sum(
        jnp.exp(xm - m_new), axis=1, keepdims=True
    )
    m_sc[...] = m_new
    am = jnp.where(mask, a, 0.0)
    d_sc[...] += jnp.sum(am * x, axis=1, keepdims=True)
    n_sc[...] += jnp.sum(am, axis=1, keepdims=True)

    @pl.when(i == TC_STEPS - 1)
    def _fin():
        m_out[...] = m_sc[...]
        s_out[...] = s_sc[...]
        d_out[...] = d_sc[...]
        n_out[...] = n_sc[...]


def _tc_partials(logits, actions):
    def imap(i):
        return (0, jnp.where(i == TC_STEPS - 1, V_SC_END // TCW, i))

    return pl.pallas_call(
        _tc_kernel,
        grid=(TC_STEPS,),
        in_specs=[
            pl.BlockSpec((B, TCW), imap),
            pl.BlockSpec((B, TCW), imap),
        ],
        out_specs=[pl.BlockSpec((B, 1), lambda i: (0, 0))] * 4,
        out_shape=[jax.ShapeDtypeStruct((B, 1), jnp.float32)] * 4,
        scratch_shapes=[pltpu.VMEM((B, 1), jnp.float32)] * 4,
    )(logits, actions)


NC, NS, L = 2, 16, 16
NW = NC * NS
TR = 8
CS = NW // TR
TPW = SC_NT // CS
CT = 16
NCH = TPW // CT
CW = CT * 128


@functools.partial(
    pl.kernel,
    out_type=[jax.ShapeDtypeStruct((NW, TR, L), jnp.float32)] * 4,
    mesh=plsc.VectorSubcoreMesh(core_axis_name="c", subcore_axis_name="s"),
    compiler_params=pltpu.CompilerParams(use_tc_tiling_on_sc=True),
    scratch_types=[
        pltpu.VMEM((TR, CW), jnp.float32),
        pltpu.VMEM((TR, CW), jnp.float32),
        pltpu.VMEM((TR, CW), jnp.float32),
        pltpu.VMEM((TR, CW), jnp.float32),
        pltpu.VMEM((TR, L), jnp.float32),
        pltpu.SemaphoreType.DMA,
        pltpu.SemaphoreType.DMA,
        pltpu.SemaphoreType.DMA,
        pltpu.SemaphoreType.DMA,
    ],
)
def _sc_partials(
    log_hbm, act_hbm, m_out, s_out, d_out, n_out,
    xb0, xb1, ab0, ab1, stage, sx0, sx1, sa0, sa1,
):
    wid = lax.axis_index("s") * NC + lax.axis_index("c")
    rtile = wid % TR
    cslab = wid // TR
    row0 = rtile * TR
    xbufs, abufs = (xb0, xb1), (ab0, ab1)
    sxs, sas = (sx0, sx1), (sa0, sa1)

    def start(k):
        col0 = (SC_T0 + cslab * TPW + k * CT) * 128
        cx = pltpu.async_copy(
            log_hbm.at[pl.ds(row0, TR), pl.ds(col0, CW)], xbufs[k % 2], sxs[k % 2]
        )
        ca = pltpu.async_copy(
            act_hbm.at[pl.ds(row0, TR), pl.ds(col0, CW)], abufs[k % 2], sas[k % 2]
        )
        return cx, ca

    m8 = [jnp.full((L,), NEG_BIG, jnp.float32) for _ in range(TR)]
    s8 = [jnp.zeros((L,), jnp.float32) for _ in range(TR)]
    d8 = [jnp.zeros((L,), jnp.float32) for _ in range(TR)]
    n8 = [jnp.zeros((L,), jnp.float32) for _ in range(TR)]

    inflight = start(0)
    for k in range(NCH):
        nxt = start(k + 1) if k + 1 < NCH else None
        inflight[0].wait()
        inflight[1].wait()
        xb, ab = xbufs[k % 2], abufs[k % 2]

        def maxbody(j, carry, xb=xb):
            return tuple(
                jnp.maximum(carry[r], xb[r, pl.ds(j * L, L)]) for r in range(TR)
            )

        m8_new = list(lax.fori_loop(0, CW // L, maxbody, tuple(m8)))
        for r in range(TR):
            s8[r] = s8[r] * jnp.exp(m8[r] - m8_new[r])
        m8 = m8_new

        def accbody(j, carry, xb=xb, ab=ab, m8=tuple(m8)):
            s, d, n = carry
            s, d, n = list(s), list(d), list(n)
            for r in range(TR):
                x = xb[r, pl.ds(j * L, L)]
                a = ab[r, pl.ds(j * L, L)]
                s[r] = s[r] + jnp.exp(x - m8[r])
                d[r] = d[r] + a * x
                n[r] = n[r] + a
            return tuple(s), tuple(d), tuple(n)

        s8, d8, n8 = lax.fori_loop(
            0, CW // L, accbody, (tuple(s8), tuple(d8), tuple(n8))
        )
        s8, d8, n8 = list(s8), list(d8), list(n8)
        inflight = nxt

    for vecs, out in ((m8, m_out), (s8, s_out), (d8, d_out), (n8, n_out)):
        for r in range(TR):
            stage[r, pl.ds(0, L)] = vecs[r]
        pltpu.sync_copy(stage, out.at[wid])


def kernel(logits, actions):
    m_tc, s_tc, d_tc, n_tc = _tc_partials(logits, actions)
    m_sc, s_sc, d_sc, n_sc = _sc_partials(logits, actions)

    def per_row(p):
        p = p.reshape(CS, TR, TR, L)
        p = p.transpose(1, 2, 0, 3).reshape(B, CS * L)
        return p

    m_l = per_row(m_sc)
    s_l = per_row(s_sc)
    d = d_tc + per_row(d_sc).sum(axis=1, keepdims=True)
    n = n_tc + per_row(n_sc).sum(axis=1, keepdims=True)

    m_all = jnp.maximum(jnp.max(m_l, axis=1, keepdims=True), m_tc)
    s_all = jnp.sum(s_l * jnp.exp(m_l - m_all), axis=1, keepdims=True) + s_tc * jnp.exp(
        m_tc - m_all
    )
    lse = m_all + jnp.log(s_all)

    from jax.scipy.special import gammaln

    rt_zero = jnp.minimum(jnp.abs(s_tc[0, 0]), jnp.float32(0.0))
    a0 = gammaln(jnp.float32(1.0) + rt_zero)
    a1 = gammaln(jnp.float32(2.0) + rt_zero)
    out = a1 + d - n * lse - (a0 * (V - n) + a1 * n)
    return out


# speedup vs baseline: 3.3346x; 1.1647x over previous
import functools

import jax
import jax.numpy as jnp
from jax import lax
from jax.experimental import pallas as pl
from jax.experimental.pallas import tpu as pltpu
from jax.experimental.pallas import tpu_sc as plsc

B, V = 64, 100000
NEG_BIG = -3.0e38

SC_T0 = 397
SC_NT = 384
V_TC = SC_T0 * 128
V_SC_END = (SC_T0 + SC_NT) * 128


TCW = 4096
TC_FULL = -(-V_TC // TCW)
TC_STEPS = TC_FULL + 1
TAIL_BLK = V_SC_END // TCW


def _tc_kernel(x_ref, a_ref, m_out, s_out, d_out, n_out, m_sc, s_sc, d_sc, n_sc):
    i = pl.program_id(0)

    @pl.when(i == 0)
    def _init():
        m_sc[...] = jnp.full_like(m_sc, NEG_BIG)
        s_sc[...] = jnp.zeros_like(s_sc)
        d_sc[...] = jnp.zeros_like(d_sc)
        n_sc[...] = jnp.zeros_like(n_sc)

    x = x_ref[...]
    a = a_ref[...]
    blk = jnp.where(i == TC_STEPS - 1, TAIL_BLK, i)
    col = blk * TCW + lax.broadcasted_iota(jnp.int32, x.shape, 1)
    lo = jnp.where(i == TC_STEPS - 1, V_SC_END, 0)
    hi = jnp.where(i == TC_STEPS - 1, V, V_TC)
    mask = (col >= lo) & (col < hi)
    xm = jnp.where(mask, x, NEG_BIG)

    m_old = m_sc[...]
    m_new = jnp.maximum(m_old, jnp.max(xm, axis=1, keepdims=True))
    s_sc[...] = s_sc[...] * jnp.exp(m_old - m_new) + jnp.sum(
        jnp.exp(xm - m_new), axis=1, keepdims=True
    )
    m_sc[...] = m_new
    am = jnp.where(mask, a, 0.0)
    d_sc[...] += jnp.sum(am * x, axis=1, keepdims=True)
    n_sc[...] += jnp.sum(am, axis=1, keepdims=True)

    @pl.when(i == TC_STEPS - 1)
    def _fin():
        m_out[...] = m_sc[...]
        s_out[...] = s_sc[...]
        d_out[...] = d_sc[...]
        n_out[...] = n_sc[...]


def _tc_partials(logits, actions):
    def imap(i):
        return (0, jnp.where(i == TC_STEPS - 1, TAIL_BLK, i))

    return pl.pallas_call(
        _tc_kernel,
        grid=(TC_STEPS,),
        in_specs=[
            pl.BlockSpec((B, TCW), imap),
            pl.BlockSpec((B, TCW), imap),
        ],
        out_specs=[pl.BlockSpec((B, 1), lambda i: (0, 0))] * 4,
        out_shape=[jax.ShapeDtypeStruct((B, 1), jnp.float32)] * 4,
        scratch_shapes=[pltpu.VMEM((B, 1), jnp.float32)] * 4,
    )(logits, actions)


NC, NS, L = 2, 16, 16
NW = NC * NS
TR = 8
CS = NW // TR
TPW = SC_NT // CS
CT = 12
NCH = TPW // CT
CW = CT * 128


@functools.partial(
    pl.kernel,
    out_type=[jax.ShapeDtypeStruct((NW * TR, L), jnp.float32)] * 4,
    mesh=plsc.VectorSubcoreMesh(core_axis_name="c", subcore_axis_name="s"),
    compiler_params=pltpu.CompilerParams(use_tc_tiling_on_sc=True),
    scratch_types=[
        pltpu.VMEM((TR, CW), jnp.float32),
        pltpu.VMEM((TR, CW), jnp.float32),
        pltpu.VMEM((TR, CW), jnp.float32),
        pltpu.VMEM((TR, CW), jnp.float32),
        pltpu.VMEM((TR, L), jnp.float32),
        pltpu.SemaphoreType.DMA,
        pltpu.SemaphoreType.DMA,
        pltpu.SemaphoreType.DMA,
        pltpu.SemaphoreType.DMA,
    ],
)
def _sc_partials(
    log_hbm, act_hbm, m_out, s_out, d_out, n_out,
    xb0, xb1, ab0, ab1, stage, sx0, sx1, sa0, sa1,
):
    wid = lax.axis_index("s") * NC + lax.axis_index("c")
    rtile = wid % TR
    cslab = wid // TR
    row0 = rtile * TR
    xbufs, abufs = (xb0, xb1), (ab0, ab1)
    sxs, sas = (sx0, sx1), (sa0, sa1)

    def start(k):
        col0 = (SC_T0 + cslab * TPW + k * CT) * 128
        cx = pltpu.async_copy(
            log_hbm.at[pl.ds(row0, TR), pl.ds(col0, CW)], xbufs[k % 2], sxs[k % 2]
        )
        ca = pltpu.async_copy(
            act_hbm.at[pl.ds(row0, TR), pl.ds(col0, CW)], abufs[k % 2], sas[k % 2]
        )
        return cx, ca

    m8 = [jnp.full((L,), NEG_BIG, jnp.float32) for _ in range(TR)]
    s8 = [jnp.zeros((L,), jnp.float32) for _ in range(TR)]
    d8 = [jnp.zeros((L,), jnp.float32) for _ in range(TR)]
    n8 = [jnp.zeros((L,), jnp.float32) for _ in range(TR)]

    inflight = start(0)
    for k in range(NCH):
        nxt = start(k + 1) if k + 1 < NCH else None
        inflight[0].wait()
        inflight[1].wait()
        xb, ab = xbufs[k % 2], abufs[k % 2]

        def maxbody(j, carry, xb=xb):
            return tuple(
                jnp.maximum(carry[r], xb[r, pl.ds(j * L, L)]) for r in range(TR)
            )

        m8_new = list(lax.fori_loop(0, CW // L, maxbody, tuple(m8)))
        for r in range(TR):
            s8[r] = s8[r] * jnp.exp(m8[r] - m8_new[r])
        m8 = m8_new

        def accbody(j, carry, xb=xb, ab=ab, m8=tuple(m8)):
            s, d, n = carry
            s, d, n = list(s), list(d), list(n)
            for r in range(TR):
                x = xb[r, pl.ds(j * L, L)]
                a = ab[r, pl.ds(j * L, L)]
                s[r] = s[r] + jnp.exp(x - m8[r])
                d[r] = d[r] + a * x
                n[r] = n[r] + a
            return tuple(s), tuple(d), tuple(n)

        s8, d8, n8 = lax.fori_loop(
            0, CW // L, accbody, (tuple(s8), tuple(d8), tuple(n8))
        )
        s8, d8, n8 = list(s8), list(d8), list(n8)
        inflight = nxt

    for vecs, out in ((m8, m_out), (s8, s_out), (d8, d_out), (n8, n_out)):
        for r in range(TR):
            stage[r, pl.ds(0, L)] = vecs[r]
        pltpu.sync_copy(stage, out.at[pl.ds(wid * TR, TR), pl.ds(0, L)])


def _merge_kernel(
    m4_ref, s4_ref, d4_ref, n4_ref, mt_ref, st_ref, dt_ref, nt_ref, a01_ref, o_ref
):
    m_all = mt_ref[...]
    mp = []
    for c in range(CS):
        p = m4_ref[pl.ds(c * B, B), :]
        mp.append(p)
        m_all = jnp.maximum(m_all, jnp.max(p, axis=1, keepdims=True))
    s_all = st_ref[...] * jnp.exp(mt_ref[...] - m_all)
    d = dt_ref[...]
    n = nt_ref[...]
    for c in range(CS):
        s_all += jnp.sum(
            s4_ref[pl.ds(c * B, B), :] * jnp.exp(mp[c] - m_all),
            axis=1, keepdims=True,
        )
        d += jnp.sum(d4_ref[pl.ds(c * B, B), :], axis=1, keepdims=True)
        n += jnp.sum(n4_ref[pl.ds(c * B, B), :], axis=1, keepdims=True)
    lse = m_all + jnp.log(s_all)
    a0 = a01_ref[0]
    a1 = a01_ref[1]
    o_ref[...] = a1 + d - n * lse - (a0 * (V - n) + a1 * n)


def _merge(m4, s4, d4, n4, m_tc, s_tc, d_tc, n_tc, a01):
    return pl.pallas_call(
        _merge_kernel,
        in_specs=[pl.BlockSpec(memory_space=pltpu.VMEM)] * 8
        + [pl.BlockSpec(memory_space=pltpu.SMEM)],
        out_specs=pl.BlockSpec(memory_space=pltpu.VMEM),
        out_shape=jax.ShapeDtypeStruct((B, 1), jnp.float32),
    )(m4, s4, d4, n4, m_tc, s_tc, d_tc, n_tc, a01)


def kernel(logits, actions):
    from jax.scipy.special import gammaln

    rt_zero = jnp.minimum(jnp.abs(logits[0, 0]), jnp.float32(0.0))
    a01 = gammaln(jnp.stack([1.0 + rt_zero, 2.0 + rt_zero]).astype(jnp.float32))

    m_tc, s_tc, d_tc, n_tc = _tc_partials(logits, actions)
    m4, s4, d4, n4 = _sc_partials(logits, actions)
    return _merge(m4, s4, d4, n4, m_tc, s_tc, d_tc, n_tc, a01)
